# Initial kernel scaffold; baseline (speedup 1.0000x reference)
import jax
import jax.numpy as jnp
from jax.experimental import pallas as pl


def kernel(x, edge_index, W1, b1, W2, b2):
    raise NotImplementedError("write your pallas kernel here")


# speedup vs baseline: 8.6774x; 8.6774x over previous
import functools

import jax
import jax.numpy as jnp
from jax import lax
from jax.experimental import pallas as pl
from jax.experimental.pallas import tpu as pltpu
from jax.experimental.pallas import tpu_sc as plsc

N = 10000
D = 128
E = 320000

NC = 2
NS = 16
NW = NC * NS
EPW = E // NW
CHUNK = 80
NCHUNK = EPW // CHUNK
NP = 10240
NPS = NP // NS

_mesh = plsc.VectorSubcoreMesh(core_axis_name="c", subcore_axis_name="s")
_sc_params = pltpu.CompilerParams(use_tc_tiling_on_sc=False,
                                 needs_layout_passes=False)


@functools.partial(
    pl.kernel,
    out_type=jax.ShapeDtypeStruct((NC * NP, 16), jnp.float32),
    mesh=_mesh,
    compiler_params=_sc_params,
    scratch_types=[
        pltpu.VMEM_SHARED((NP, 16), jnp.float32),
        pltpu.VMEM((CHUNK,), jnp.int32),
        pltpu.VMEM((CHUNK, 16), jnp.float32),
        pltpu.VMEM((NPS, 16), jnp.float32),
    ],
)
def _deg_kernel(col_hbm, zeros_hbm, ones_hbm, out_hbm, degp, cidx, ones, zbuf):
    cid = lax.axis_index("c")
    sid = lax.axis_index("s")
    wid = cid * NS + sid
    pltpu.sync_copy(zeros_hbm, zbuf)
    pltpu.sync_copy(ones_hbm, ones)
    pltpu.sync_copy(zbuf, degp.at[pl.ds(sid * NPS, NPS)])
    plsc.subcore_barrier()

    def body(i, carry):
        base = wid * EPW + i * CHUNK
        pltpu.sync_copy(col_hbm.at[pl.ds(base, CHUNK)], cidx)
        pltpu.sync_copy(ones, degp.at[cidx], add=True)
        return carry

    lax.fori_loop(0, NCHUNK, body, 0)
    plsc.subcore_barrier()
    pltpu.sync_copy(degp.at[pl.ds(sid * NPS, NPS)],
                    out_hbm.at[pl.ds(cid * NP + sid * NPS, NPS)])


@functools.partial(
    pl.kernel,
    out_type=jax.ShapeDtypeStruct((NC * NP, D), jnp.float32),
    mesh=_mesh,
    compiler_params=_sc_params,
    scratch_types=[
        pltpu.VMEM_SHARED((NP, D), jnp.float32),
        pltpu.VMEM((CHUNK,), jnp.int32),
        pltpu.VMEM((CHUNK,), jnp.int32),
        pltpu.VMEM((CHUNK, D), jnp.float32),
        pltpu.VMEM((CHUNK, D), jnp.float32),
        pltpu.SemaphoreType.DMA,
    ],
)
def _agg_kernel(y_hbm, row_hbm, col_hbm, zeros_hbm, out_hbm,
                acc, ridx, cidx, rows, zbuf, sem):
    cid = lax.axis_index("c")
    sid = lax.axis_index("s")
    wid = cid * NS + sid
    pltpu.sync_copy(zeros_hbm, zbuf)
    for j in range(NPS // CHUNK):
        pltpu.sync_copy(zbuf, acc.at[pl.ds(sid * NPS + j * CHUNK, CHUNK)])
    plsc.subcore_barrier()

    def body(i, carry):
        base = wid * EPW + i * CHUNK
        pltpu.sync_copy(row_hbm.at[pl.ds(base, CHUNK)], ridx)
        pltpu.sync_copy(col_hbm.at[pl.ds(base, CHUNK)], cidx)
        pltpu.async_copy(y_hbm.at[ridx], rows, sem).wait()
        pltpu.sync_copy(rows, acc.at[cidx], add=True)
        return carry

    lax.fori_loop(0, NCHUNK, body, 0)
    plsc.subcore_barrier()
    pltpu.sync_copy(acc.at[pl.ds(sid * NPS, NPS)],
                    out_hbm.at[pl.ds(cid * NP + sid * NPS, NPS)])


_EU = 4


@functools.partial(
    pl.kernel,
    out_type=jax.ShapeDtypeStruct((E,), jnp.float32),
    mesh=_mesh,
    compiler_params=_sc_params,
    scratch_types=[
        pltpu.VMEM((CHUNK,), jnp.int32),
        pltpu.VMEM((CHUNK,), jnp.int32),
        pltpu.VMEM((CHUNK, D), jnp.float32),
        pltpu.VMEM((CHUNK, D), jnp.float32),
        pltpu.VMEM((CHUNK,), jnp.float32),
        pltpu.SemaphoreType.DMA,
    ],
)
def _edge_kernel(h_hbm, row_hbm, col_hbm, out_hbm,
                 ridx, cidx, rr, rc, sbuf, sem):
    cid = lax.axis_index("c")
    sid = lax.axis_index("s")
    wid = cid * NS + sid
    def chunk_body(i, carry):
        base = wid * EPW + i * CHUNK
        pltpu.sync_copy(row_hbm.at[pl.ds(base, CHUNK)], ridx)
        pltpu.sync_copy(col_hbm.at[pl.ds(base, CHUNK)], cidx)
        cp1 = pltpu.async_copy(h_hbm.at[ridx], rr, sem)
        cp2 = pltpu.async_copy(h_hbm.at[cidx], rc, sem)
        cp1.wait()
        cp2.wait()

        lane = lax.iota(jnp.int32, 16)

        def group_body(g, carry2):
            def edge_body(t, acc16):
                for u in range(_EU):
                    e = g * 16 + t * _EU + u
                    s = jnp.zeros((16,), jnp.float32)
                    for k in range(D // 16):
                        a = rr[e, pl.ds(k * 16, 16)]
                        b = rc[e, pl.ds(k * 16, 16)]
                        s = s + a * b
                    acc16 = jnp.where(lane == t * _EU + u, jnp.sum(s), acc16)
                return acc16

            z = lax.fori_loop(0, 16 // _EU, edge_body,
                              jnp.zeros((16,), jnp.float32))
            sbuf[pl.ds(g * 16, 16)] = 1.0 / (1.0 + jnp.exp(-z))
            return carry2

        lax.fori_loop(0, CHUNK // 16, group_body, 0)
        pltpu.sync_copy(sbuf, out_hbm.at[pl.ds(base, CHUNK)])
        return carry

    lax.fori_loop(0, NCHUNK, chunk_body, 0)


def _dis_from(degp_ref):
    deg = degp_ref[:N, 0:1] + degp_ref[NP:NP + N, 0:1]
    safe = jnp.where(deg > 0, deg, 1.0)
    return jnp.where(deg > 0, lax.rsqrt(safe), 0.0)


def _tc1_body(x_ref, w_ref, degp_ref, y_ref):
    dis = _dis_from(degp_ref)
    y_ref[...] = jnp.dot(x_ref[...], w_ref[...],
                         preferred_element_type=jnp.float32) * dis


def _tc2_body(agg_ref, degp_ref, b_ref, w_ref, y_ref):
    dis = _dis_from(degp_ref)
    agg = agg_ref[:N] + agg_ref[NP:NP + N]
    h = jnp.maximum(agg * dis + b_ref[...][None, :], 0.0)
    y_ref[...] = jnp.dot(h, w_ref[...],
                         preferred_element_type=jnp.float32) * dis


def _tc3_body(agg_ref, degp_ref, b_ref, h_ref):
    dis = _dis_from(degp_ref)
    agg = agg_ref[:N] + agg_ref[NP:NP + N]
    h_ref[...] = agg * dis + b_ref[...][None, :]


_tc1 = pl.pallas_call(_tc1_body,
                      out_shape=jax.ShapeDtypeStruct((N, D), jnp.float32))
_tc2 = pl.pallas_call(_tc2_body,
                      out_shape=jax.ShapeDtypeStruct((N, D), jnp.float32))
_tc3 = pl.pallas_call(_tc3_body,
                      out_shape=jax.ShapeDtypeStruct((N, D), jnp.float32))


def kernel(x, edge_index, W1, b1, W2, b2):
    row = edge_index[0].astype(jnp.int32)
    col = edge_index[1].astype(jnp.int32)
    zeros16 = jnp.zeros((NPS, 16), jnp.float32)
    ones16 = jnp.ones((CHUNK, 16), jnp.float32)
    zerosD = jnp.zeros((CHUNK, D), jnp.float32)

    degp = _deg_kernel(col, zeros16, ones16)
    y1 = _tc1(x, W1, degp)
    agg1 = _agg_kernel(y1, row, col, zerosD)
    y2 = _tc2(agg1, degp, b1, W2)
    agg2 = _agg_kernel(y2, row, col, zerosD)
    h2 = _tc3(agg2, degp, b2)
    return _edge_kernel(h2, row, col)


# speedup vs baseline: 16.3917x; 1.8890x over previous
import functools

import jax
import jax.numpy as jnp
from jax import lax
from jax.experimental import pallas as pl
from jax.experimental.pallas import tpu as pltpu
from jax.experimental.pallas import tpu_sc as plsc

N = 10000
D = 128
E = 320000

NC = 2
NS = 16
NW = NC * NS
EPW = E // NW
CHUNK = 80
NCHUNK = EPW // CHUNK
NP = 10240
NPS = NP // NS

_mesh = plsc.VectorSubcoreMesh(core_axis_name="c", subcore_axis_name="s")
_sc_params = pltpu.CompilerParams(use_tc_tiling_on_sc=False,
                                 needs_layout_passes=False)


@functools.partial(
    pl.kernel,
    out_type=jax.ShapeDtypeStruct((NC * NP, 16), jnp.float32),
    mesh=_mesh,
    compiler_params=_sc_params,
    scratch_types=[
        pltpu.VMEM_SHARED((NP, 16), jnp.float32),
        pltpu.VMEM((CHUNK,), jnp.int32),
        pltpu.VMEM((CHUNK,), jnp.int32),
        pltpu.VMEM((CHUNK, 16), jnp.float32),
        pltpu.VMEM((NPS, 16), jnp.float32),
        pltpu.SemaphoreType.DMA,
        pltpu.SemaphoreType.DMA,
    ],
)
def _deg_kernel(col_hbm, zeros_hbm, ones_hbm, out_hbm, degp,
                cidx0, cidx1, ones, zbuf, isem0, isem1):
    cid = lax.axis_index("c")
    sid = lax.axis_index("s")
    wid = cid * NS + sid
    e0 = wid * EPW
    cidx = (cidx0, cidx1)
    isem = (isem0, isem1)
    pltpu.sync_copy(zeros_hbm, zbuf)
    pltpu.sync_copy(ones_hbm, ones)
    pltpu.sync_copy(zbuf, degp.at[pl.ds(sid * NPS, NPS)])
    plsc.subcore_barrier()

    pltpu.async_copy(col_hbm.at[pl.ds(e0, CHUNK)], cidx0, isem0)
    pltpu.async_copy(col_hbm.at[pl.ds(e0 + CHUNK, CHUNK)], cidx1, isem1)
    last = NCHUNK - 1

    def body(io, carry):
        for b in range(2):
            i = io * 2 + b
            pltpu.make_async_copy(col_hbm.at[pl.ds(e0 + i * CHUNK, CHUNK)],
                                  cidx[b], isem[b]).wait()
            pltpu.sync_copy(ones, degp.at[cidx[b]], add=True)
            nxt = lax.min(i + 2, last) * CHUNK + e0
            pltpu.async_copy(col_hbm.at[pl.ds(nxt, CHUNK)], cidx[b], isem[b])
        return carry

    lax.fori_loop(0, (NCHUNK - 1) // 2, body, 0)
    i = NCHUNK - 1
    pltpu.make_async_copy(col_hbm.at[pl.ds(e0 + i * CHUNK, CHUNK)],
                          cidx[i % 2], isem[i % 2]).wait()
    pltpu.sync_copy(ones, degp.at[cidx[i % 2]], add=True)
    pltpu.make_async_copy(col_hbm.at[pl.ds(e0, CHUNK)],
                          cidx[(i + 1) % 2], isem[(i + 1) % 2]).wait()
    plsc.subcore_barrier()
    pltpu.sync_copy(degp.at[pl.ds(sid * NPS, NPS)],
                    out_hbm.at[pl.ds(cid * NP + sid * NPS, NPS)])


@functools.partial(
    pl.kernel,
    out_type=jax.ShapeDtypeStruct((NC * NP, D), jnp.float32),
    mesh=_mesh,
    compiler_params=_sc_params,
    scratch_types=[
        pltpu.VMEM_SHARED((NP, D), jnp.float32),
        pltpu.VMEM((CHUNK,), jnp.int32),
        pltpu.VMEM((CHUNK,), jnp.int32),
        pltpu.VMEM((CHUNK,), jnp.int32),
        pltpu.VMEM((CHUNK,), jnp.int32),
        pltpu.VMEM((CHUNK, D), jnp.float32),
        pltpu.VMEM((CHUNK, D), jnp.float32),
        pltpu.VMEM((CHUNK, D), jnp.float32),
        pltpu.SemaphoreType.DMA,
        pltpu.SemaphoreType.DMA,
        pltpu.SemaphoreType.DMA,
        pltpu.SemaphoreType.DMA,
    ],
)
def _agg_kernel(y_hbm, row_hbm, col_hbm, zeros_hbm, out_hbm,
                acc, ridx0, ridx1, cidx0, cidx1, rows0, rows1, zbuf,
                isem0, isem1, gsem0, gsem1):
    cid = lax.axis_index("c")
    sid = lax.axis_index("s")
    wid = cid * NS + sid
    e0 = wid * EPW
    ridx = (ridx0, ridx1)
    cidx = (cidx0, cidx1)
    rows = (rows0, rows1)
    isem = (isem0, isem1)
    gsem = (gsem0, gsem1)
    last = NCHUNK - 1
    pltpu.sync_copy(zeros_hbm, zbuf)
    for j in range(NPS // CHUNK):
        pltpu.sync_copy(zbuf, acc.at[pl.ds(sid * NPS + j * CHUNK, CHUNK)])
    plsc.subcore_barrier()

    def idx_start(i, b):
        pltpu.async_copy(row_hbm.at[pl.ds(e0 + i * CHUNK, CHUNK)],
                         ridx[b], isem[b])
        pltpu.async_copy(col_hbm.at[pl.ds(e0 + i * CHUNK, CHUNK)],
                         cidx[b], isem[b])

    def idx_wait(b):
        pltpu.make_async_copy(row_hbm.at[pl.ds(e0, CHUNK)],
                              ridx[b], isem[b]).wait()
        pltpu.make_async_copy(col_hbm.at[pl.ds(e0, CHUNK)],
                              cidx[b], isem[b]).wait()

    def gather_start(b):
        pltpu.async_copy(y_hbm.at[ridx[b]], rows[b], gsem[b])

    def gather_wait(b):
        pltpu.make_async_copy(y_hbm.at[ridx[b]], rows[b], gsem[b]).wait()

    idx_start(0, 0)
    idx_start(1, 1)
    idx_wait(0)
    gather_start(0)

    def body(io, carry):
        for b in range(2):
            i = io * 2 + b
            gather_wait(b)
            idx_wait(1 - b)
            gather_start(1 - b)
            pltpu.sync_copy(rows[b], acc.at[cidx[b]], add=True)
            nxt = lax.min(i + 2, last)
            idx_start(nxt, b)
        return carry

    lax.fori_loop(0, (NCHUNK - 1) // 2, body, 0)
    i = NCHUNK - 1
    b = i % 2
    gather_wait(b)
    pltpu.sync_copy(rows[b], acc.at[cidx[b]], add=True)
    idx_wait(1 - b)
    plsc.subcore_barrier()
    pltpu.sync_copy(acc.at[pl.ds(sid * NPS, NPS)],
                    out_hbm.at[pl.ds(cid * NP + sid * NPS, NPS)])


_EU = 4


@functools.partial(
    pl.kernel,
    out_type=jax.ShapeDtypeStruct((E,), jnp.float32),
    mesh=_mesh,
    compiler_params=_sc_params,
    scratch_types=[
        pltpu.VMEM((CHUNK,), jnp.int32),
        pltpu.VMEM((CHUNK,), jnp.int32),
        pltpu.VMEM((CHUNK,), jnp.int32),
        pltpu.VMEM((CHUNK,), jnp.int32),
        pltpu.VMEM((CHUNK, D), jnp.float32),
        pltpu.VMEM((CHUNK, D), jnp.float32),
        pltpu.VMEM((CHUNK, D), jnp.float32),
        pltpu.VMEM((CHUNK, D), jnp.float32),
        pltpu.VMEM((CHUNK,), jnp.float32),
        pltpu.VMEM((CHUNK,), jnp.float32),
        pltpu.SemaphoreType.DMA,
        pltpu.SemaphoreType.DMA,
        pltpu.SemaphoreType.DMA,
        pltpu.SemaphoreType.DMA,
        pltpu.SemaphoreType.DMA,
        pltpu.SemaphoreType.DMA,
    ],
)
def _edge_kernel(h_hbm, row_hbm, col_hbm, out_hbm,
                 ridx0, ridx1, cidx0, cidx1, rr0, rr1, rc0, rc1,
                 sbuf0, sbuf1, isem0, isem1, gsem0, gsem1, osem0, osem1):
    cid = lax.axis_index("c")
    sid = lax.axis_index("s")
    wid = cid * NS + sid
    e0 = wid * EPW
    ridx = (ridx0, ridx1)
    cidx = (cidx0, cidx1)
    rr = (rr0, rr1)
    rc = (rc0, rc1)
    sbuf = (sbuf0, sbuf1)
    isem = (isem0, isem1)
    gsem = (gsem0, gsem1)
    osem = (osem0, osem1)
    last = NCHUNK - 1
    lane = lax.iota(jnp.int32, 16)

    def idx_start(i, b):
        pltpu.async_copy(row_hbm.at[pl.ds(e0 + i * CHUNK, CHUNK)],
                         ridx[b], isem[b])
        pltpu.async_copy(col_hbm.at[pl.ds(e0 + i * CHUNK, CHUNK)],
                         cidx[b], isem[b])

    def idx_wait(b):
        pltpu.make_async_copy(row_hbm.at[pl.ds(e0, CHUNK)],
                              ridx[b], isem[b]).wait()
        pltpu.make_async_copy(col_hbm.at[pl.ds(e0, CHUNK)],
                              cidx[b], isem[b]).wait()

    def gather_start(b):
        pltpu.async_copy(h_hbm.at[ridx[b]], rr[b], gsem[b])
        pltpu.async_copy(h_hbm.at[cidx[b]], rc[b], gsem[b])

    def gather_wait(b):
        pltpu.make_async_copy(h_hbm.at[ridx[b]], rr[b], gsem[b]).wait()
        pltpu.make_async_copy(h_hbm.at[cidx[b]], rc[b], gsem[b]).wait()

    def compute(b):
        def group_body(g, carry2):
            def edge_body(t, acc16):
                for u in range(_EU):
                    e = g * 16 + t * _EU + u
                    s = jnp.zeros((16,), jnp.float32)
                    for k in range(D // 16):
                        av = rr[b][e, pl.ds(k * 16, 16)]
                        bv = rc[b][e, pl.ds(k * 16, 16)]
                        s = s + av * bv
                    acc16 = jnp.where(lane == t * _EU + u, jnp.sum(s), acc16)
                return acc16

            z = lax.fori_loop(0, 16 // _EU, edge_body,
                              jnp.zeros((16,), jnp.float32))
            sbuf[b][pl.ds(g * 16, 16)] = 1.0 / (1.0 + jnp.exp(-z))
            return carry2

        lax.fori_loop(0, CHUNK // 16, group_body, 0)

    idx_start(0, 0)
    idx_start(1, 1)
    idx_wait(0)
    gather_start(0)

    def chunk_body(io, carry):
        for b in range(2):
            i = io * 2 + b
            gather_wait(b)
            idx_wait(1 - b)
            gather_start(1 - b)
            compute(b)
            pltpu.async_copy(sbuf[b], out_hbm.at[pl.ds(e0 + i * CHUNK, CHUNK)],
                             osem[b])
            nxt = lax.min(i + 2, last)
            idx_start(nxt, b)
            pltpu.make_async_copy(sbuf[b], out_hbm.at[pl.ds(e0, CHUNK)],
                                  osem[b]).wait()
        return carry

    lax.fori_loop(0, (NCHUNK - 1) // 2, chunk_body, 0)
    i = NCHUNK - 1
    b = i % 2
    gather_wait(b)
    compute(b)
    pltpu.sync_copy(sbuf[b], out_hbm.at[pl.ds(e0 + i * CHUNK, CHUNK)])
    idx_wait(1 - b)


def _dis_from(degp_ref):
    deg = degp_ref[:N, 0:1] + degp_ref[NP:NP + N, 0:1]
    safe = jnp.where(deg > 0, deg, 1.0)
    return jnp.where(deg > 0, lax.rsqrt(safe), 0.0)


def _tc1_body(x_ref, w_ref, degp_ref, y_ref):
    dis = _dis_from(degp_ref)
    y_ref[...] = jnp.dot(x_ref[...], w_ref[...],
                         preferred_element_type=jnp.float32) * dis


def _tc2_body(agg_ref, degp_ref, b_ref, w_ref, y_ref):
    dis = _dis_from(degp_ref)
    agg = agg_ref[:N] + agg_ref[NP:NP + N]
    h = jnp.maximum(agg * dis + b_ref[...][None, :], 0.0)
    y_ref[...] = jnp.dot(h, w_ref[...],
                         preferred_element_type=jnp.float32) * dis


def _tc3_body(agg_ref, degp_ref, b_ref, h_ref):
    dis = _dis_from(degp_ref)
    agg = agg_ref[:N] + agg_ref[NP:NP + N]
    h_ref[...] = agg * dis + b_ref[...][None, :]


_tc1 = pl.pallas_call(_tc1_body,
                      out_shape=jax.ShapeDtypeStruct((N, D), jnp.float32))
_tc2 = pl.pallas_call(_tc2_body,
                      out_shape=jax.ShapeDtypeStruct((N, D), jnp.float32))
_tc3 = pl.pallas_call(_tc3_body,
                      out_shape=jax.ShapeDtypeStruct((N, D), jnp.float32))


def kernel(x, edge_index, W1, b1, W2, b2):
    row = edge_index[0].astype(jnp.int32)
    col = edge_index[1].astype(jnp.int32)
    zeros16 = jnp.zeros((NPS, 16), jnp.float32)
    ones16 = jnp.ones((CHUNK, 16), jnp.float32)
    zerosD = jnp.zeros((CHUNK, D), jnp.float32)

    degp = _deg_kernel(col, zeros16, ones16)
    y1 = _tc1(x, W1, degp)
    agg1 = _agg_kernel(y1, row, col, zerosD)
    y2 = _tc2(agg1, degp, b1, W2)
    agg2 = _agg_kernel(y2, row, col, zerosD)
    h2 = _tc3(agg2, degp, b2)
    return _edge_kernel(h2, row, col)


# speedup vs baseline: 18.5320x; 1.1306x over previous
import functools

import jax
import jax.numpy as jnp
from jax import lax
from jax.experimental import pallas as pl
from jax.experimental.pallas import tpu as pltpu
from jax.experimental.pallas import tpu_sc as plsc

N = 10000
D = 128
E = 320000

NC = 2
NS = 16
NW = NC * NS
EPW = E // NW
CHUNK = 80
NCHUNK = EPW // CHUNK
NP = 10240
NPS = NP // NS
RING = 3
PEEL = NCHUNK % RING

_mesh = plsc.VectorSubcoreMesh(core_axis_name="c", subcore_axis_name="s")
_sc_params = pltpu.CompilerParams(use_tc_tiling_on_sc=False,
                                 needs_layout_passes=False)


def _dma_sems():
    return [pltpu.SemaphoreType.DMA for _ in range(RING)]


@functools.partial(
    pl.kernel,
    out_type=jax.ShapeDtypeStruct((NC * NP, 16), jnp.float32),
    mesh=_mesh,
    compiler_params=_sc_params,
    scratch_types=[
        pltpu.VMEM_SHARED((NP, 16), jnp.float32),
        [pltpu.VMEM((CHUNK,), jnp.int32) for _ in range(RING)],
        pltpu.VMEM((CHUNK, 16), jnp.float32),
        pltpu.VMEM((NPS, 16), jnp.float32),
        _dma_sems(),
    ],
)
def _deg_kernel(col_hbm, zeros_hbm, ones_hbm, out_hbm,
                degp, cidx, ones, zbuf, isem):
    cid = lax.axis_index("c")
    sid = lax.axis_index("s")
    wid = cid * NS + sid
    e0 = wid * EPW
    last = NCHUNK - 1
    pltpu.sync_copy(zeros_hbm, zbuf)
    pltpu.sync_copy(ones_hbm, ones)
    pltpu.sync_copy(zbuf, degp.at[pl.ds(sid * NPS, NPS)])
    plsc.subcore_barrier()

    def idx_start(i, s):
        pltpu.async_copy(col_hbm.at[pl.ds(e0 + i * CHUNK, CHUNK)],
                         cidx[s], isem[s])

    def idx_wait(s):
        pltpu.make_async_copy(col_hbm.at[pl.ds(e0, CHUNK)],
                              cidx[s], isem[s]).wait()

    for s in range(RING):
        idx_start(s, s)

    def step(i, s):
        idx_wait(s)
        pltpu.sync_copy(ones, degp.at[cidx[s]], add=True)
        idx_start(lax.min(i + RING, last), s)

    def body(io, carry):
        for b in range(RING):
            step(io * RING + b, b)
        return carry

    lax.fori_loop(0, (NCHUNK - PEEL) // RING, body, 0)
    for p in range(PEEL):
        i = NCHUNK - PEEL + p
        s = i % RING
        idx_wait(s)
        pltpu.sync_copy(ones, degp.at[cidx[s]], add=True)
    for p in range(PEEL, RING):
        idx_wait((NCHUNK - PEEL + p) % RING)
    plsc.subcore_barrier()
    pltpu.sync_copy(degp.at[pl.ds(sid * NPS, NPS)],
                    out_hbm.at[pl.ds(cid * NP + sid * NPS, NPS)])


@functools.partial(
    pl.kernel,
    out_type=jax.ShapeDtypeStruct((NC * NP, D), jnp.float32),
    mesh=_mesh,
    compiler_params=_sc_params,
    scratch_types=[
        pltpu.VMEM_SHARED((NP, D), jnp.float32),
        [pltpu.VMEM((CHUNK,), jnp.int32) for _ in range(RING)],
        [pltpu.VMEM((CHUNK,), jnp.int32) for _ in range(RING)],
        [pltpu.VMEM((CHUNK, D), jnp.float32) for _ in range(RING)],
        pltpu.VMEM((CHUNK, D), jnp.float32),
        _dma_sems(),
        _dma_sems(),
    ],
)
def _agg_kernel(y_hbm, row_hbm, col_hbm, zeros_hbm, out_hbm,
                acc, ridx, cidx, rows, zbuf, isem, gsem):
    cid = lax.axis_index("c")
    sid = lax.axis_index("s")
    wid = cid * NS + sid
    e0 = wid * EPW
    last = NCHUNK - 1
    pltpu.sync_copy(zeros_hbm, zbuf)
    for j in range(NPS // CHUNK):
        pltpu.sync_copy(zbuf, acc.at[pl.ds(sid * NPS + j * CHUNK, CHUNK)])
    plsc.subcore_barrier()

    def idx_start(i, s):
        pltpu.async_copy(row_hbm.at[pl.ds(e0 + i * CHUNK, CHUNK)],
                         ridx[s], isem[s])
        pltpu.async_copy(col_hbm.at[pl.ds(e0 + i * CHUNK, CHUNK)],
                         cidx[s], isem[s])

    def idx_wait(s):
        pltpu.make_async_copy(row_hbm.at[pl.ds(e0, CHUNK)],
                              ridx[s], isem[s]).wait()
        pltpu.make_async_copy(col_hbm.at[pl.ds(e0, CHUNK)],
                              cidx[s], isem[s]).wait()

    def gather_start(s):
        pltpu.async_copy(y_hbm.at[ridx[s]], rows[s], gsem[s])

    def gather_wait(s):
        pltpu.make_async_copy(y_hbm.at[ridx[s]], rows[s], gsem[s]).wait()

    for s in range(RING):
        idx_start(s, s)
    for s in range(RING - 1):
        idx_wait(s)
        gather_start(s)

    def step(i, s):
        s1 = (s + RING - 1) % RING
        gather_wait(s)
        idx_wait(s1)
        gather_start(s1)
        pltpu.sync_copy(rows[s], acc.at[cidx[s]], add=True)
        idx_start(lax.min(i + RING, last), s)

    def body(io, carry):
        for b in range(RING):
            step(io * RING + b, b)
        return carry

    lax.fori_loop(0, (NCHUNK - PEEL) // RING, body, 0)
    for p in range(PEEL):
        i = NCHUNK - PEEL + p
        s = i % RING
        gather_wait(s)
        pltpu.sync_copy(rows[s], acc.at[cidx[s]], add=True)
    for p in range(PEEL, RING):
        idx_wait((NCHUNK - PEEL + p) % RING)
    plsc.subcore_barrier()
    pltpu.sync_copy(acc.at[pl.ds(sid * NPS, NPS)],
                    out_hbm.at[pl.ds(cid * NP + sid * NPS, NPS)])


_EU = 4


@functools.partial(
    pl.kernel,
    out_type=jax.ShapeDtypeStruct((E,), jnp.float32),
    mesh=_mesh,
    compiler_params=_sc_params,
    scratch_types=[
        [pltpu.VMEM((CHUNK,), jnp.int32) for _ in range(RING)],
        [pltpu.VMEM((CHUNK,), jnp.int32) for _ in range(RING)],
        [pltpu.VMEM((CHUNK, D), jnp.float32) for _ in range(RING)],
        [pltpu.VMEM((CHUNK, D), jnp.float32) for _ in range(RING)],
        [pltpu.VMEM((CHUNK,), jnp.float32) for _ in range(RING)],
        _dma_sems(),
        _dma_sems(),
        _dma_sems(),
    ],
)
def _edge_kernel(h_hbm, row_hbm, col_hbm, out_hbm,
                 ridx, cidx, rr, rc, sbuf, isem, gsem, osem):
    cid = lax.axis_index("c")
    sid = lax.axis_index("s")
    wid = cid * NS + sid
    e0 = wid * EPW
    last = NCHUNK - 1
    lane = lax.iota(jnp.int32, 16)

    def idx_start(i, s):
        pltpu.async_copy(row_hbm.at[pl.ds(e0 + i * CHUNK, CHUNK)],
                         ridx[s], isem[s])
        pltpu.async_copy(col_hbm.at[pl.ds(e0 + i * CHUNK, CHUNK)],
                         cidx[s], isem[s])

    def idx_wait(s):
        pltpu.make_async_copy(row_hbm.at[pl.ds(e0, CHUNK)],
                              ridx[s], isem[s]).wait()
        pltpu.make_async_copy(col_hbm.at[pl.ds(e0, CHUNK)],
                              cidx[s], isem[s]).wait()

    def gather_start(s):
        pltpu.async_copy(h_hbm.at[ridx[s]], rr[s], gsem[s])
        pltpu.async_copy(h_hbm.at[cidx[s]], rc[s], gsem[s])

    def gather_wait(s):
        pltpu.make_async_copy(h_hbm.at[ridx[s]], rr[s], gsem[s]).wait()
        pltpu.make_async_copy(h_hbm.at[cidx[s]], rc[s], gsem[s]).wait()

    def compute(s):
        def group_body(g, carry2):
            def edge_body(t, acc16):
                for u in range(_EU):
                    e = g * 16 + t * _EU + u
                    v = jnp.zeros((16,), jnp.float32)
                    for k in range(D // 16):
                        av = rr[s][e, pl.ds(k * 16, 16)]
                        bv = rc[s][e, pl.ds(k * 16, 16)]
                        v = v + av * bv
                    acc16 = jnp.where(lane == t * _EU + u, jnp.sum(v), acc16)
                return acc16

            z = lax.fori_loop(0, 16 // _EU, edge_body,
                              jnp.zeros((16,), jnp.float32))
            sbuf[s][pl.ds(g * 16, 16)] = 1.0 / (1.0 + jnp.exp(-z))
            return carry2

        lax.fori_loop(0, CHUNK // 16, group_body, 0)

    for s in range(RING):
        idx_start(s, s)
    for s in range(RING - 1):
        idx_wait(s)
        gather_start(s)

    def step(i, s):
        s1 = (s + RING - 1) % RING
        gather_wait(s)
        idx_wait(s1)
        gather_start(s1)
        compute(s)
        pltpu.async_copy(sbuf[s], out_hbm.at[pl.ds(e0 + i * CHUNK, CHUNK)],
                         osem[s])
        idx_start(lax.min(i + RING, last), s)
        pltpu.make_async_copy(sbuf[s], out_hbm.at[pl.ds(e0, CHUNK)],
                              osem[s]).wait()

    def body(io, carry):
        for b in range(RING):
            step(io * RING + b, b)
        return carry

    lax.fori_loop(0, (NCHUNK - PEEL) // RING, body, 0)
    for p in range(PEEL):
        i = NCHUNK - PEEL + p
        s = i % RING
        gather_wait(s)
        compute(s)
        pltpu.sync_copy(sbuf[s], out_hbm.at[pl.ds(e0 + i * CHUNK, CHUNK)])
    for p in range(PEEL, RING):
        idx_wait((NCHUNK - PEEL + p) % RING)


def _dis_from(degp_ref):
    deg = degp_ref[:N, 0:1] + degp_ref[NP:NP + N, 0:1]
    safe = jnp.where(deg > 0, deg, 1.0)
    return jnp.where(deg > 0, lax.rsqrt(safe), 0.0)


def _tc1_body(x_ref, w_ref, degp_ref, y_ref):
    dis = _dis_from(degp_ref)
    y_ref[...] = jnp.dot(x_ref[...], w_ref[...],
                         preferred_element_type=jnp.float32) * dis


def _tc2_body(agg_ref, degp_ref, b_ref, w_ref, y_ref):
    dis = _dis_from(degp_ref)
    agg = agg_ref[:N] + agg_ref[NP:NP + N]
    h = jnp.maximum(agg * dis + b_ref[...][None, :], 0.0)
    y_ref[...] = jnp.dot(h, w_ref[...],
                         preferred_element_type=jnp.float32) * dis


def _tc3_body(agg_ref, degp_ref, b_ref, h_ref):
    dis = _dis_from(degp_ref)
    agg = agg_ref[:N] + agg_ref[NP:NP + N]
    h_ref[...] = agg * dis + b_ref[...][None, :]


_tc1 = pl.pallas_call(_tc1_body,
                      out_shape=jax.ShapeDtypeStruct((N, D), jnp.float32))
_tc2 = pl.pallas_call(_tc2_body,
                      out_shape=jax.ShapeDtypeStruct((N, D), jnp.float32))
_tc3 = pl.pallas_call(_tc3_body,
                      out_shape=jax.ShapeDtypeStruct((N, D), jnp.float32))


def kernel(x, edge_index, W1, b1, W2, b2):
    row = edge_index[0].astype(jnp.int32)
    col = edge_index[1].astype(jnp.int32)
    zeros16 = jnp.zeros((NPS, 16), jnp.float32)
    ones16 = jnp.ones((CHUNK, 16), jnp.float32)
    zerosD = jnp.zeros((CHUNK, D), jnp.float32)

    degp = _deg_kernel(col, zeros16, ones16)
    y1 = _tc1(x, W1, degp)
    agg1 = _agg_kernel(y1, row, col, zerosD)
    y2 = _tc2(agg1, degp, b1, W2)
    agg2 = _agg_kernel(y2, row, col, zerosD)
    h2 = _tc3(agg2, degp, b2)
    return _edge_kernel(h2, row, col)


# speedup vs baseline: 18.7072x; 1.0095x over previous
import functools

import jax
import jax.numpy as jnp
from jax import lax
from jax.experimental import pallas as pl
from jax.experimental.pallas import tpu as pltpu
from jax.experimental.pallas import tpu_sc as plsc

N = 10000
D = 128
E = 320000

NC = 2
NS = 16
NW = NC * NS
EPW = E // NW
CHUNK = 80
NCHUNK = EPW // CHUNK
NP = 10240
NPS = NP // NS
RING = 3
PEEL = NCHUNK % RING

_mesh = plsc.VectorSubcoreMesh(core_axis_name="c", subcore_axis_name="s")
_sc_params = pltpu.CompilerParams(use_tc_tiling_on_sc=False,
                                 needs_layout_passes=False)


def _dma_sems():
    return [pltpu.SemaphoreType.DMA for _ in range(RING)]


@functools.partial(
    pl.kernel,
    out_type=jax.ShapeDtypeStruct((NC * NP, 16), jnp.float32),
    mesh=_mesh,
    compiler_params=_sc_params,
    scratch_types=[
        pltpu.VMEM_SHARED((NP, 16), jnp.float32),
        [pltpu.VMEM((CHUNK,), jnp.int32) for _ in range(RING)],
        pltpu.VMEM((CHUNK, 16), jnp.float32),
        pltpu.VMEM((NPS, 16), jnp.float32),
        _dma_sems(),
    ],
)
def _deg_kernel(col_hbm, zeros_hbm, ones_hbm, out_hbm,
                degp, cidx, ones, zbuf, isem):
    cid = lax.axis_index("c")
    sid = lax.axis_index("s")
    wid = cid * NS + sid
    e0 = wid * EPW
    last = NCHUNK - 1
    pltpu.sync_copy(zeros_hbm, zbuf)
    pltpu.sync_copy(ones_hbm, ones)
    pltpu.sync_copy(zbuf, degp.at[pl.ds(sid * NPS, NPS)])
    plsc.subcore_barrier()

    def idx_start(i, s):
        pltpu.async_copy(col_hbm.at[pl.ds(e0 + i * CHUNK, CHUNK)],
                         cidx[s], isem[s])

    def idx_wait(s):
        pltpu.make_async_copy(col_hbm.at[pl.ds(e0, CHUNK)],
                              cidx[s], isem[s]).wait()

    for s in range(RING):
        idx_start(s, s)

    def step(i, s):
        idx_wait(s)
        pltpu.sync_copy(ones, degp.at[cidx[s]], add=True)
        idx_start(lax.min(i + RING, last), s)

    def body(io, carry):
        for b in range(RING):
            step(io * RING + b, b)
        return carry

    lax.fori_loop(0, (NCHUNK - PEEL) // RING, body, 0)
    for p in range(PEEL):
        i = NCHUNK - PEEL + p
        s = i % RING
        idx_wait(s)
        pltpu.sync_copy(ones, degp.at[cidx[s]], add=True)
    for p in range(PEEL, RING):
        idx_wait((NCHUNK - PEEL + p) % RING)
    plsc.subcore_barrier()
    pltpu.sync_copy(degp.at[pl.ds(sid * NPS, NPS)],
                    out_hbm.at[pl.ds(cid * NP + sid * NPS, NPS)])


@functools.partial(
    pl.kernel,
    out_type=jax.ShapeDtypeStruct((NC * NP, D), jnp.float32),
    mesh=_mesh,
    compiler_params=_sc_params,
    scratch_types=[
        pltpu.VMEM_SHARED((NP, D), jnp.float32),
        [pltpu.VMEM((CHUNK,), jnp.int32) for _ in range(RING)],
        [pltpu.VMEM((CHUNK,), jnp.int32) for _ in range(RING)],
        [pltpu.VMEM((CHUNK, D), jnp.float32) for _ in range(RING)],
        pltpu.VMEM((CHUNK, D), jnp.float32),
        _dma_sems(),
        _dma_sems(),
    ],
)
def _agg_kernel(y_hbm, row_hbm, col_hbm, zeros_hbm, out_hbm,
                acc, ridx, cidx, rows, zbuf, isem, gsem):
    cid = lax.axis_index("c")
    sid = lax.axis_index("s")
    wid = cid * NS + sid
    e0 = wid * EPW
    last = NCHUNK - 1
    pltpu.sync_copy(zeros_hbm, zbuf)
    for j in range(NPS // CHUNK):
        pltpu.sync_copy(zbuf, acc.at[pl.ds(sid * NPS + j * CHUNK, CHUNK)])
    plsc.subcore_barrier()

    def idx_start(i, s):
        pltpu.async_copy(row_hbm.at[pl.ds(e0 + i * CHUNK, CHUNK)],
                         ridx[s], isem[s])
        pltpu.async_copy(col_hbm.at[pl.ds(e0 + i * CHUNK, CHUNK)],
                         cidx[s], isem[s])

    def idx_wait(s):
        pltpu.make_async_copy(row_hbm.at[pl.ds(e0, CHUNK)],
                              ridx[s], isem[s]).wait()
        pltpu.make_async_copy(col_hbm.at[pl.ds(e0, CHUNK)],
                              cidx[s], isem[s]).wait()

    def gather_start(s):
        pltpu.async_copy(y_hbm.at[ridx[s]], rows[s], gsem[s])

    def gather_wait(s):
        pltpu.make_async_copy(y_hbm.at[ridx[s]], rows[s], gsem[s]).wait()

    for s in range(RING):
        idx_start(s, s)
    for s in range(RING - 1):
        idx_wait(s)
        gather_start(s)

    def step(i, s):
        s1 = (s + RING - 1) % RING
        gather_wait(s)
        idx_wait(s1)
        gather_start(s1)
        pltpu.sync_copy(rows[s], acc.at[cidx[s]], add=True)
        idx_start(lax.min(i + RING, last), s)

    def body(io, carry):
        for b in range(RING):
            step(io * RING + b, b)
        return carry

    lax.fori_loop(0, (NCHUNK - PEEL) // RING, body, 0)
    for p in range(PEEL):
        i = NCHUNK - PEEL + p
        s = i % RING
        gather_wait(s)
        pltpu.sync_copy(rows[s], acc.at[cidx[s]], add=True)
    for p in range(PEEL, RING):
        idx_wait((NCHUNK - PEEL + p) % RING)
    plsc.subcore_barrier()
    pltpu.sync_copy(acc.at[pl.ds(sid * NPS, NPS)],
                    out_hbm.at[pl.ds(cid * NP + sid * NPS, NPS)])


_EU = 4
RING_E = 2
PEEL_E = NCHUNK % RING_E


@functools.partial(
    pl.kernel,
    out_type=jax.ShapeDtypeStruct((E,), jnp.float32),
    mesh=_mesh,
    compiler_params=_sc_params,
    scratch_types=[
        pltpu.VMEM_SHARED((NP, D), jnp.float32),
        [pltpu.VMEM((CHUNK,), jnp.int32) for _ in range(RING_E)],
        [pltpu.VMEM((CHUNK,), jnp.int32) for _ in range(RING_E)],
        [pltpu.VMEM((CHUNK, D), jnp.float32) for _ in range(RING_E)],
        [pltpu.VMEM((CHUNK, D), jnp.float32) for _ in range(RING_E)],
        [pltpu.VMEM((CHUNK,), jnp.float32) for _ in range(RING_E)],
        [pltpu.SemaphoreType.DMA for _ in range(RING_E)],
        [pltpu.SemaphoreType.DMA for _ in range(RING_E)],
        [pltpu.SemaphoreType.DMA for _ in range(RING_E)],
    ],
)
def _edge_kernel(h_hbm, row_hbm, col_hbm, out_hbm,
                 hsh, ridx, cidx, rr, rc, sbuf, isem, gsem, osem):
    cid = lax.axis_index("c")
    sid = lax.axis_index("s")
    wid = cid * NS + sid
    e0 = wid * EPW
    last = NCHUNK - 1
    lane = lax.iota(jnp.int32, 16)
    pltpu.sync_copy(h_hbm.at[pl.ds(sid * NPS, NPS)],
                    hsh.at[pl.ds(sid * NPS, NPS)])
    plsc.subcore_barrier()

    def idx_start(i, s):
        pltpu.async_copy(row_hbm.at[pl.ds(e0 + i * CHUNK, CHUNK)],
                         ridx[s], isem[s])
        pltpu.async_copy(col_hbm.at[pl.ds(e0 + i * CHUNK, CHUNK)],
                         cidx[s], isem[s])

    def idx_wait(s):
        pltpu.make_async_copy(row_hbm.at[pl.ds(e0, CHUNK)],
                              ridx[s], isem[s]).wait()
        pltpu.make_async_copy(col_hbm.at[pl.ds(e0, CHUNK)],
                              cidx[s], isem[s]).wait()

    def gather_start(s):
        pltpu.async_copy(hsh.at[ridx[s]], rr[s], gsem[s])
        pltpu.async_copy(hsh.at[cidx[s]], rc[s], gsem[s])

    def gather_wait(s):
        pltpu.make_async_copy(hsh.at[ridx[s]], rr[s], gsem[s]).wait()
        pltpu.make_async_copy(hsh.at[cidx[s]], rc[s], gsem[s]).wait()

    def compute(s):
        def group_body(g, carry2):
            def edge_body(t, acc16):
                for u in range(_EU):
                    e = g * 16 + t * _EU + u
                    v = jnp.zeros((16,), jnp.float32)
                    for k in range(D // 16):
                        av = rr[s][e, pl.ds(k * 16, 16)]
                        bv = rc[s][e, pl.ds(k * 16, 16)]
                        v = v + av * bv
                    acc16 = jnp.where(lane == t * _EU + u, jnp.sum(v), acc16)
                return acc16

            z = lax.fori_loop(0, 16 // _EU, edge_body,
                              jnp.zeros((16,), jnp.float32))
            sbuf[s][pl.ds(g * 16, 16)] = 1.0 / (1.0 + jnp.exp(-z))
            return carry2

        lax.fori_loop(0, CHUNK // 16, group_body, 0)

    for s in range(RING_E):
        idx_start(s, s)
    idx_wait(0)
    gather_start(0)

    def step(i, s):
        s1 = 1 - s
        gather_wait(s)
        idx_wait(s1)
        gather_start(s1)
        compute(s)
        pltpu.async_copy(sbuf[s], out_hbm.at[pl.ds(e0 + i * CHUNK, CHUNK)],
                         osem[s])
        idx_start(lax.min(i + RING_E, last), s)
        pltpu.make_async_copy(sbuf[s], out_hbm.at[pl.ds(e0, CHUNK)],
                              osem[s]).wait()

    def body(io, carry):
        for b in range(RING_E):
            step(io * RING_E + b, b)
        return carry

    lax.fori_loop(0, (NCHUNK - PEEL_E) // RING_E, body, 0)
    for p in range(PEEL_E):
        i = NCHUNK - PEEL_E + p
        s = i % RING_E
        gather_wait(s)
        compute(s)
        pltpu.sync_copy(sbuf[s], out_hbm.at[pl.ds(e0 + i * CHUNK, CHUNK)])
    for p in range(PEEL_E, RING_E):
        idx_wait((NCHUNK - PEEL_E + p) % RING_E)


def _dis_from(degp_ref):
    deg = degp_ref[:N, 0:1] + degp_ref[NP:NP + N, 0:1]
    safe = jnp.where(deg > 0, deg, 1.0)
    return jnp.where(deg > 0, lax.rsqrt(safe), 0.0)


def _tc_mm_body(x_ref, w_ref, y_ref):
    y_ref[...] = jnp.dot(x_ref[...], w_ref[...],
                         preferred_element_type=jnp.float32)


def _tc1_body(xw_ref, degp_ref, y_ref):
    y_ref[...] = xw_ref[...] * _dis_from(degp_ref)


def _tc2_body(agg_ref, degp_ref, b_ref, w_ref, y_ref):
    dis = _dis_from(degp_ref)
    agg = agg_ref[:N] + agg_ref[NP:NP + N]
    h = jnp.maximum(agg * dis + b_ref[...][None, :], 0.0)
    y_ref[...] = jnp.dot(h, w_ref[...],
                         preferred_element_type=jnp.float32) * dis


def _tc3_body(agg_ref, degp_ref, b_ref, h_ref):
    dis = _dis_from(degp_ref)
    agg = agg_ref[:N] + agg_ref[NP:NP + N]
    h_ref[:N] = agg * dis + b_ref[...][None, :]
    h_ref[N:] = jnp.zeros((NP - N, D), jnp.float32)


_tc_mm = pl.pallas_call(_tc_mm_body,
                        out_shape=jax.ShapeDtypeStruct((N, D), jnp.float32))
_tc1 = pl.pallas_call(_tc1_body,
                      out_shape=jax.ShapeDtypeStruct((N, D), jnp.float32))
_tc2 = pl.pallas_call(_tc2_body,
                      out_shape=jax.ShapeDtypeStruct((N, D), jnp.float32))
_tc3 = pl.pallas_call(_tc3_body,
                      out_shape=jax.ShapeDtypeStruct((NP, D), jnp.float32))


def kernel(x, edge_index, W1, b1, W2, b2):
    row = edge_index[0].astype(jnp.int32)
    col = edge_index[1].astype(jnp.int32)
    zeros16 = jnp.zeros((NPS, 16), jnp.float32)
    ones16 = jnp.ones((CHUNK, 16), jnp.float32)
    zerosD = jnp.zeros((CHUNK, D), jnp.float32)

    degp = _deg_kernel(col, zeros16, ones16)
    xw1 = _tc_mm(x, W1)
    y1 = _tc1(xw1, degp)
    agg1 = _agg_kernel(y1, row, col, zerosD)
    y2 = _tc2(agg1, degp, b1, W2)
    agg2 = _agg_kernel(y2, row, col, zerosD)
    h2 = _tc3(agg2, degp, b2)
    return _edge_kernel(h2, row, col)
